# baseline (speedup 1.0000x reference)
import functools

import jax
import jax.numpy as jnp
from jax import lax
from jax.experimental import pallas as pl
from jax.experimental.pallas import tpu as pltpu
from jax.experimental.pallas import tpu_sc as plsc

_E = 64
_D = 768
_DFF = 768
_T = 2048
_C = 64
_NW = 32
_TPW = _T // _NW
_EPS = 2
_NSTEP = _E // _EPS + 2
_EO_ROWS = _NSTEP * _EPS * _C
_SPW = _EO_ROWS // _NW
_TPAD = _T + 8


def _gate_body(x_ref, wg_ref, sx_ref, slot_ref, tm_ref):
    x = x_ref[...]
    logits = jnp.dot(x, wg_ref[...], preferred_element_type=jnp.float32)
    m = jnp.max(logits, axis=1, keepdims=True)
    e = jnp.exp(logits - m)
    p = e / jnp.sum(e, axis=1, keepdims=True)
    gate = jnp.max(p, axis=1, keepdims=True)
    lane = lax.broadcasted_iota(jnp.int32, p.shape, 1)
    idx = jnp.min(jnp.where(p == gate, lane, _E), axis=1, keepdims=True)
    onehot = (lane == idx).astype(jnp.float32)

    nb = _T // 128
    oh3 = onehot.astype(jnp.bfloat16).reshape(nb, 128, _E)
    r = lax.broadcasted_iota(jnp.int32, (128, 128), 0)
    c = lax.broadcasted_iota(jnp.int32, (128, 128), 1)
    tril = (r >= c).astype(jnp.bfloat16)
    totals = jnp.sum(onehot.reshape(nb, 128, _E), axis=1)
    rb = lax.broadcasted_iota(jnp.int32, (nb, nb), 0)
    cb = lax.broadcasted_iota(jnp.int32, (nb, nb), 1)
    stril = (rb > cb).astype(jnp.float32)
    offs = jnp.dot(stril, totals, preferred_element_type=jnp.float32,
                   precision=lax.Precision.HIGHEST)
    blocks = []
    for b in range(nb):
        incl = jnp.dot(tril, oh3[b], preferred_element_type=jnp.float32)
        blocks.append(incl + offs[b:b + 1, :])
    locations = jnp.concatenate(blocks, axis=0) - 1.0

    keepm = onehot * (locations < float(_C)).astype(jnp.float32)
    kept = jnp.sum(keepm, axis=1, keepdims=True)
    loc1 = jnp.sum(locations * keepm, axis=1, keepdims=True)
    slot_f = idx.astype(jnp.float32) * float(_C) + loc1
    slot_ref[...] = jnp.where(kept > 0, slot_f, float(_E * _C)).astype(jnp.int32)

    loc_lane = lax.broadcasted_iota(jnp.int32, (_T, _C), 1).astype(jnp.float32)
    loc1hot = (loc_lane == loc1).astype(jnp.float32) * kept
    tvals = (lax.broadcasted_iota(jnp.int32, (_T, 1), 0) + 1).astype(jnp.float32)
    a = lax.dot_general(onehot, loc1hot * tvals,
                        dimension_numbers=(((0,), (0,)), ((), ())),
                        preferred_element_type=jnp.float32,
                        precision=lax.Precision.HIGHEST)
    tm = jnp.where(a > 0.5, a - 1.0, float(_T)).astype(jnp.int32)
    tm_ref[...] = jnp.concatenate(
        [tm, jnp.full((4, _C), _T, jnp.int32)], axis=0)

    sx = x * (gate * kept)
    sx_ref[...] = jnp.concatenate(
        [sx, jnp.zeros((_TPAD - _T, _D), jnp.float32)], axis=0)


def _gate(tokens, Wg):
    return pl.pallas_call(
        _gate_body,
        out_shape=(jax.ShapeDtypeStruct((_TPAD, _D), jnp.float32),
                   jax.ShapeDtypeStruct((_T, 1), jnp.int32),
                   jax.ShapeDtypeStruct((_E + 4, _C), jnp.int32)),
    )(tokens, Wg)


def _mlp_body(disp_ref, w1_ref, w2_ref, out_ref):
    e = pl.program_id(0)

    @pl.when(e < _E // _EPS)
    def _():
        x = disp_ref[...]
        for k in range(_EPS):
            xk = x[k * _C:(k + 1) * _C]
            h = jnp.maximum(
                jnp.dot(xk, w1_ref[k], preferred_element_type=jnp.float32), 0.0)
            out_ref[k * _C:(k + 1) * _C, :] = jnp.dot(
                h, w2_ref[k], preferred_element_type=jnp.float32)

    @pl.when(e >= _E // _EPS)
    def _():
        out_ref[...] = jnp.zeros_like(out_ref)


def _mlp(disp, W1, W2):
    nmax = _E // _EPS - 1
    return pl.pallas_call(
        _mlp_body,
        grid=(_NSTEP,),
        in_specs=[
            pl.BlockSpec((_EPS * _C, _D), lambda e: (e, 0)),
            pl.BlockSpec((_EPS, _D, _DFF), lambda e: (jnp.minimum(e, nmax), 0, 0)),
            pl.BlockSpec((_EPS, _DFF, _D), lambda e: (jnp.minimum(e, nmax), 0, 0)),
        ],
        out_specs=pl.BlockSpec((_EPS * _C, _D), lambda e: (e, 0)),
        out_shape=jax.ShapeDtypeStruct((_EO_ROWS, _D), jnp.float32),
    )(disp, W1, W2)


@functools.cache
def _sc_kernels():
    mesh = plsc.VectorSubcoreMesh(core_axis_name="c", subcore_axis_name="s")

    @functools.partial(
        pl.kernel,
        out_type=jax.ShapeDtypeStruct((_EO_ROWS, _D), jnp.float32),
        mesh=mesh,
        scratch_types=[pltpu.VMEM((_SPW,), jnp.int32),
                       pltpu.VMEM((_SPW, _D), jnp.float32),
                       pltpu.SemaphoreType.DMA],
    )
    def dispatch_sc(sx_hbm, tm_hbm, disp_hbm, idx_v, rows_v, sem):
        wid = lax.axis_index("s") * 2 + lax.axis_index("c")
        base = wid * _SPW
        pltpu.sync_copy(tm_hbm.at[pl.ds(wid * _SPW, _SPW)], idx_v)
        pltpu.async_copy(sx_hbm.at[idx_v], rows_v, sem).wait()
        pltpu.sync_copy(rows_v, disp_hbm.at[pl.ds(base, _SPW)])

    @functools.partial(
        pl.kernel,
        out_type=jax.ShapeDtypeStruct((_T, _D), jnp.float32),
        mesh=mesh,
        scratch_types=[pltpu.VMEM((_TPW,), jnp.int32),
                       pltpu.VMEM((_TPW, _D), jnp.float32),
                       pltpu.SemaphoreType.DMA],
    )
    def combine_sc(eo_hbm, slot_hbm, out_hbm, idx_v, rows_v, sem):
        wid = lax.axis_index("s") * 2 + lax.axis_index("c")
        base = wid * _TPW
        pltpu.sync_copy(slot_hbm.at[wid], idx_v)
        pltpu.async_copy(eo_hbm.at[idx_v], rows_v, sem).wait()
        pltpu.sync_copy(rows_v, out_hbm.at[pl.ds(base, _TPW)])

    return dispatch_sc, combine_sc


def kernel(hidden_states, Wg, W1, W2):
    B, S, D = hidden_states.shape
    tokens = jnp.transpose(hidden_states, (1, 0, 2)).reshape(S * B, D)
    dispatch_sc, combine_sc = _sc_kernels()
    sx, slot, tm = _gate(tokens, Wg)
    slot_w = slot.reshape(_NW, _TPW)
    tm_w = tm.reshape(_EO_ROWS)
    disp = dispatch_sc(sx, tm_w)
    eo = _mlp(disp, W1, W2)
    out_tok = combine_sc(eo, slot_w)
    return jnp.transpose(out_tok.reshape(S, B, D), (1, 0, 2))


# speedup vs baseline: 2.2294x; 2.2294x over previous
import functools

import jax
import jax.numpy as jnp
from jax import lax
from jax.experimental import pallas as pl
from jax.experimental.pallas import tpu as pltpu
from jax.experimental.pallas import tpu_sc as plsc

_E = 64
_D = 768
_DFF = 768
_T = 2048
_C = 64
_NW = 32
_TPW = _T // _NW
_EPS = 2
_NSTEP = _E // _EPS + 2
_EO_ROWS = _NSTEP * _EPS * _C
_SPW = _E * _C // _NW
_TPAD = _T + 8


def _gate_body(x_ref, wg_ref, sx_ref, slot_ref, tm_ref):
    x = x_ref[...]
    logits = jnp.dot(x, wg_ref[...], preferred_element_type=jnp.float32)
    m = jnp.max(logits, axis=1, keepdims=True)
    e = jnp.exp(logits - m)
    p = e / jnp.sum(e, axis=1, keepdims=True)
    gate = jnp.max(p, axis=1, keepdims=True)
    lane = lax.broadcasted_iota(jnp.int32, p.shape, 1)
    idx = jnp.min(jnp.where(p == gate, lane, _E), axis=1, keepdims=True)
    onehot = (lane == idx).astype(jnp.float32)

    nb = _T // 128
    oh3 = onehot.astype(jnp.bfloat16).reshape(nb, 128, _E)
    r = lax.broadcasted_iota(jnp.int32, (128, 128), 0)
    c = lax.broadcasted_iota(jnp.int32, (128, 128), 1)
    tril = (r >= c).astype(jnp.bfloat16)
    totals = jnp.sum(onehot.reshape(nb, 128, _E), axis=1)
    rb = lax.broadcasted_iota(jnp.int32, (nb, nb), 0)
    cb = lax.broadcasted_iota(jnp.int32, (nb, nb), 1)
    stril = (rb > cb).astype(jnp.float32)
    offs = jnp.dot(stril, totals, preferred_element_type=jnp.float32,
                   precision=lax.Precision.HIGHEST)
    blocks = []
    for b in range(nb):
        incl = jnp.dot(tril, oh3[b], preferred_element_type=jnp.float32)
        blocks.append(incl + offs[b:b + 1, :])
    locations = jnp.concatenate(blocks, axis=0) - 1.0

    keepm = onehot * (locations < float(_C)).astype(jnp.float32)
    kept = jnp.sum(keepm, axis=1, keepdims=True)
    loc1 = jnp.sum(locations * keepm, axis=1, keepdims=True)
    slot_f = idx.astype(jnp.float32) * float(_C) + loc1
    slot_ref[...] = jnp.where(kept > 0, slot_f, float(_E * _C)).astype(jnp.int32)

    loc_lane = lax.broadcasted_iota(jnp.int32, (_T, _C), 1).astype(jnp.float32)
    loc1hot = (loc_lane == loc1).astype(jnp.float32) * kept
    tvals = (lax.broadcasted_iota(jnp.int32, (_T, 1), 0) + 1).astype(jnp.float32)
    a = lax.dot_general(onehot, loc1hot * tvals,
                        dimension_numbers=(((0,), (0,)), ((), ())),
                        preferred_element_type=jnp.float32,
                        precision=lax.Precision.HIGHEST)
    er = lax.broadcasted_iota(jnp.int32, (_E, _C), 0)
    ec = lax.broadcasted_iota(jnp.int32, (_E, _C), 1)
    spread = (er * _C + ec) % _T
    tm_ref[...] = jnp.where(a > 0.5, a - 1.0,
                            spread.astype(jnp.float32)).astype(jnp.int32)

    sx = x * (gate * kept)
    sx_ref[...] = jnp.concatenate(
        [sx, jnp.zeros((_TPAD - _T, _D), jnp.float32)], axis=0)


def _gate(tokens, Wg):
    return pl.pallas_call(
        _gate_body,
        out_shape=(jax.ShapeDtypeStruct((_TPAD, _D), jnp.float32),
                   jax.ShapeDtypeStruct((_T, 1), jnp.int32),
                   jax.ShapeDtypeStruct((_E, _C), jnp.int32)),
    )(tokens, Wg)


def _mlp_body(disp_ref, w1_ref, w2_ref, out_ref):
    e = pl.program_id(0)

    @pl.when(e < _E // _EPS)
    def _():
        x = disp_ref[...]
        for k in range(_EPS):
            xk = x[k * _C:(k + 1) * _C]
            h = jnp.maximum(
                jnp.dot(xk, w1_ref[k], preferred_element_type=jnp.float32), 0.0)
            out_ref[k * _C:(k + 1) * _C, :] = jnp.dot(
                h, w2_ref[k], preferred_element_type=jnp.float32)

    @pl.when(e >= _E // _EPS)
    def _():
        out_ref[...] = jnp.zeros_like(out_ref)


def _mlp(disp, W1, W2):
    nmax = _E // _EPS - 1
    return pl.pallas_call(
        _mlp_body,
        grid=(_NSTEP,),
        in_specs=[
            pl.BlockSpec((_EPS * _C, _D), lambda e: (e, 0)),
            pl.BlockSpec((_EPS, _D, _DFF), lambda e: (jnp.minimum(e, nmax), 0, 0)),
            pl.BlockSpec((_EPS, _DFF, _D), lambda e: (jnp.minimum(e, nmax), 0, 0)),
        ],
        out_specs=pl.BlockSpec((_EPS * _C, _D), lambda e: (e, 0)),
        out_shape=jax.ShapeDtypeStruct((_EO_ROWS, _D), jnp.float32),
    )(disp, W1, W2)


@functools.cache
def _sc_kernels():
    mesh = plsc.VectorSubcoreMesh(core_axis_name="c", subcore_axis_name="s")

    @functools.partial(
        pl.kernel,
        out_type=jax.ShapeDtypeStruct((_EO_ROWS, _D), jnp.float32),
        mesh=mesh,
        scratch_types=[pltpu.VMEM((_SPW,), jnp.int32),
                       pltpu.VMEM((_SPW, _D), jnp.float32),
                       pltpu.SemaphoreType.DMA],
    )
    def dispatch_sc(sx_hbm, tm_hbm, disp_hbm, idx_v, rows_v, sem):
        wid = lax.axis_index("s") * 2 + lax.axis_index("c")
        base = wid * _SPW
        pltpu.sync_copy(tm_hbm.at[wid], idx_v)
        pltpu.async_copy(sx_hbm.at[idx_v], rows_v, sem).wait()
        pltpu.sync_copy(rows_v, disp_hbm.at[pl.ds(base, _SPW)])

    @functools.partial(
        pl.kernel,
        out_type=jax.ShapeDtypeStruct((_T, _D), jnp.float32),
        mesh=mesh,
        scratch_types=[pltpu.VMEM((_TPW,), jnp.int32),
                       pltpu.VMEM((_TPW, _D), jnp.float32),
                       pltpu.SemaphoreType.DMA],
    )
    def combine_sc(eo_hbm, slot_hbm, out_hbm, idx_v, rows_v, sem):
        wid = lax.axis_index("s") * 2 + lax.axis_index("c")
        base = wid * _TPW
        pltpu.sync_copy(slot_hbm.at[wid], idx_v)
        pltpu.async_copy(eo_hbm.at[idx_v], rows_v, sem).wait()
        pltpu.sync_copy(rows_v, out_hbm.at[pl.ds(base, _TPW)])

    return dispatch_sc, combine_sc


def kernel(hidden_states, Wg, W1, W2):
    B, S, D = hidden_states.shape
    tokens = jnp.transpose(hidden_states, (1, 0, 2)).reshape(S * B, D)
    dispatch_sc, combine_sc = _sc_kernels()
    sx, slot, tm = _gate(tokens, Wg)
    slot_w = slot.reshape(_NW, _TPW)
    tm_w = tm.reshape(_NW, _SPW)
    disp = dispatch_sc(sx, tm_w)
    eo = _mlp(disp, W1, W2)
    out_tok = combine_sc(eo, slot_w)
    return jnp.transpose(out_tok.reshape(S, B, D), (1, 0, 2))
